# Initial kernel scaffold; baseline (speedup 1.0000x reference)
import jax
import jax.numpy as jnp
from jax.experimental import pallas as pl


def kernel(input_data, edge_index, W1, att_src1, att_dst1, b1, W2, att_src2, att_dst2, b2):
    raise NotImplementedError("write your pallas kernel here")


# speedup vs baseline: 2.4173x; 2.4173x over previous
import functools

import jax
import jax.numpy as jnp
from jax.experimental import pallas as pl


def _lrelu(v):
    return jnp.where(v > 0, v, 0.2 * v)


def _gat_math(input_data, edge_index, W1, att_src1, att_dst1, b1,
              W2, att_src2, att_dst2, b2):
    x = input_data
    n, d_in = x.shape
    h1 = att_src1.shape[1]
    hid = att_src1.shape[2]
    out_ch = att_src2.shape[2]
    src = edge_index[0]
    dst = edge_index[1]

    W1r = W1.reshape(d_in, h1, hid)
    wa_src = jnp.einsum('dhc,hc->dh', W1r, att_src1[0])
    wa_dst = jnp.einsum('dhc,hc->dh', W1r, att_dst1[0])
    s = x @ jnp.concatenate([wa_src, wa_dst], axis=1)
    a_src = s[:, :h1]
    a_dst = s[:, h1:]

    w = jnp.exp(_lrelu(a_src[src] + a_dst[dst]))
    w_self = jnp.exp(_lrelu(a_src + a_dst))
    denom = jax.ops.segment_sum(w, dst, num_segments=n) + w_self

    outs = []
    xg = x[src]
    for hh in range(h1):
        z = jax.ops.segment_sum(w[:, hh][:, None] * xg, dst, num_segments=n)
        z = z + w_self[:, hh][:, None] * x
        z = z / denom[:, hh][:, None]
        outs.append(z @ W1r[:, hh, :])
    x2 = jnp.concatenate(outs, axis=1) + b1
    x2 = jax.nn.elu(x2)

    h2 = x2 @ W2
    a2s = h2 @ att_src2[0, 0]
    a2d = h2 @ att_dst2[0, 0]
    w2 = jnp.exp(_lrelu(a2s[src] + a2d[dst]))
    w2_self = jnp.exp(_lrelu(a2s + a2d))
    den2 = jax.ops.segment_sum(w2, dst, num_segments=n) + w2_self
    z2 = jax.ops.segment_sum(w2[:, None] * h2[src], dst, num_segments=n)
    z2 = z2 + w2_self[:, None] * h2
    return z2, den2


def _finish_kernel(z2_ref, den2_ref, b2_ref, out_ref):
    out_ref[...] = z2_ref[...] / den2_ref[...] + b2_ref[...]


def _finish(z2, den2, b2):
    n, out_ch = z2.shape
    blk = 1000
    return pl.pallas_call(
        _finish_kernel,
        grid=(n // blk,),
        in_specs=[
            pl.BlockSpec((blk, out_ch), lambda i: (i, 0)),
            pl.BlockSpec((blk, 1), lambda i: (i, 0)),
            pl.BlockSpec((1, out_ch), lambda i: (0, 0)),
        ],
        out_specs=pl.BlockSpec((blk, out_ch), lambda i: (i, 0)),
        out_shape=jax.ShapeDtypeStruct((n, out_ch), z2.dtype),
    )(z2, den2[:, None], b2[None, :])


@jax.jit
def kernel(input_data, edge_index, W1, att_src1, att_dst1, b1,
           W2, att_src2, att_dst2, b2):
    z2, den2 = _gat_math(input_data, edge_index, W1, att_src1, att_dst1, b1,
                         W2, att_src2, att_dst2, b2)
    return _finish(z2, den2, b2)


# speedup vs baseline: 4.4067x; 1.8230x over previous
import functools

import jax
import jax.numpy as jnp
from jax import lax
from jax.experimental import pallas as pl
from jax.experimental.pallas import tpu as pltpu
from jax.experimental.pallas import tpu_sc as plsc

N = 10000
E = 160000
D_IN = 500
HID = 1000
H1 = 8
OUT = 200

CH = 256
BK = 25
NB = N // BK
NBP = 416
NW = 32
EPW = 5120
EP = NW * EPW
CK3 = 128
CW = 512
EPP = EP + 4096

_i16 = functools.partial(lax.iota, jnp.int32)


@functools.cache
def _mesh():
    return plsc.VectorSubcoreMesh(core_axis_name="c", subcore_axis_name="s")


_SC_PARAMS = pltpu.CompilerParams(use_tc_tiling_on_sc=False)


def _wid():
    return lax.axis_index("s") * 2 + lax.axis_index("c")


def _lrelu(v):
    return jnp.where(v > 0, v, 0.2 * v)


def _perm(v, idx):
    return v.at[idx].get(mode="promise_in_bounds")


def _splat(v, j):
    return _perm(v, jnp.full((16,), j, dtype=jnp.int32))


def _bucket(dv):
    return (dv.astype(jnp.float32) * jnp.float32(1.0 / BK)
            + jnp.float32(5e-5)).astype(jnp.int32)


def _sch_body(dstg, cnto, dstb, cntv, smc):
    wid = _wid()
    pltpu.sync_copy(dstg.at[pl.ds(wid * EPW, EPW)], dstb)

    def zs(i, c):
        smc[i] = 0
        return c
    lax.fori_loop(0, NBP, zs, 0)

    def hist(k, c):
        bv = _bucket(dstb[pl.ds(k * 16, 16)])
        for j in range(16):
            b = bv[j]
            smc[b] = smc[b] + 1
        return c
    lax.fori_loop(0, EPW // 16, hist, 0)

    def tov(k, c):
        v = jnp.zeros((16,), jnp.int32)
        for j in range(16):
            v = jnp.where(_i16(16) == j,
                          jnp.full((16,), smc[k * 16 + j], jnp.int32), v)
        cntv[pl.ds(k * 16, 16)] = v
        return c
    lax.fori_loop(0, NBP // 16, tov, 0)
    pltpu.sync_copy(cntv, cnto.at[wid])


def _sc_h(dstg):
    return pl.kernel(
        _sch_body,
        out_type=jax.ShapeDtypeStruct((NW, NBP), jnp.int32),
        mesh=_mesh(),
        compiler_params=_SC_PARAMS,
        scratch_types=[
            pltpu.VMEM((EPW,), jnp.int32),
            pltpu.VMEM((NBP,), jnp.int32),
            pltpu.SMEM((NBP,), jnp.int32),
        ],
    )(dstg)


def _tcp_body(cnt_ref, baset_ref, starts_ref):
    c = cnt_ref[...].astype(jnp.float32)
    ti = lax.broadcasted_iota(jnp.int32, (NW, NW), 0)
    tj = lax.broadcasted_iota(jnp.int32, (NW, NW), 1)
    tri_w = (tj < ti).astype(jnp.float32)
    base_part = jnp.dot(tri_w, c, preferred_element_type=jnp.float32)
    tot = jnp.sum(c, axis=0, keepdims=True)
    tot8 = jnp.floor((tot + 7.0) * 0.125) * 8.0
    bi = lax.broadcasted_iota(jnp.int32, (NBP, NBP), 0)
    bj = lax.broadcasted_iota(jnp.int32, (NBP, NBP), 1)
    tri_b = (bi < bj).astype(jnp.float32)
    starts = jnp.dot(tot8, tri_b, preferred_element_type=jnp.float32)
    baset_ref[...] = (base_part + starts).astype(jnp.int32)
    st = jnp.concatenate([starts, tot], axis=0)
    starts_ref[...] = jnp.broadcast_to(st, (4, 2, NBP)).reshape(
        8, NBP).astype(jnp.int32)


def _tc_p(cnt):
    return pl.pallas_call(
        _tcp_body,
        grid=(1,),
        in_specs=[pl.BlockSpec((NW, NBP), lambda i: (0, 0))],
        out_specs=[
            pl.BlockSpec((NW, NBP), lambda i: (0, 0)),
            pl.BlockSpec((8, NBP), lambda i: (0, 0)),
        ],
        out_shape=[
            jax.ShapeDtypeStruct((NW, NBP), jnp.int32),
            jax.ShapeDtypeStruct((8, NBP), jnp.int32),
        ],
    )(cnt)


def _sc2_body(stab, srcg, dstg, baset, reco, srcso,
              srcb, dstb, dstc, bb, srows, drows, recb, posb, basev,
              smb, smc, sem):
    wid = _wid()
    base = wid * EPW
    iota = _i16(16)
    rot8 = (iota + 8) & 15
    sh2 = (iota - 2) & 15

    pltpu.sync_copy(srcg.at[pl.ds(base, EPW)], srcb)
    pltpu.sync_copy(dstg.at[pl.ds(base, EPW)], dstb)
    pltpu.sync_copy(baset.at[wid], basev)

    def prep(k, c):
        dv = dstb[pl.ds(k * 16, 16)]
        bb[pl.ds(k * 16, 16)] = _bucket(dv)
        dstc[pl.ds(k * 16, 16)] = jnp.minimum(dv, N - 1)
        return c
    lax.fori_loop(0, EPW // 16, prep, 0)

    def tosm(k, c):
        v = basev[pl.ds(k * 16, 16)]
        for j in range(16):
            smb[k * 16 + j] = v[j]
            smc[k * 16 + j] = 0
        return c
    lax.fori_loop(0, NBP // 16, tosm, 0)

    def chunk(cc, c):
        cb = cc * CW
        g1 = pltpu.async_copy(stab.at[srcb.at[pl.ds(cb, CW)]], srows, sem)
        g1.wait()
        g2 = pltpu.async_copy(stab.at[dstc.at[pl.ds(cb, CW)]], drows, sem)
        g2.wait()

        def build(k2, c2):
            srcv = srcb[pl.ds(cb + k2 * 16, 16)]
            dstv = dstb[pl.ds(cb + k2 * 16, 16)]
            bvv = bb[pl.ds(cb + k2 * 16, 16)]
            pv = jnp.zeros((16,), jnp.int32)
            for j in range(16):
                el = k2 * 16 + j
                srow = srows[el, :]
                drow = drows[el, :]
                v = srow + _perm(drow, rot8)
                w = jnp.exp(_lrelu(v))
                wi = lax.bitcast_convert_type(_perm(w, sh2), jnp.int32)
                row = jnp.where(iota == 0, jnp.full((16,), srcv[j], jnp.int32),
                                jnp.where(iota == 1,
                                          jnp.full((16,), dstv[j], jnp.int32),
                                          wi))
                recb[el, :] = row
                b = bvv[j]
                cnt = smc[b]
                smc[b] = cnt + 1
                pv = jnp.where(iota == j,
                               jnp.full((16,), smb[b] + cnt, jnp.int32), pv)
            posb[(cb >> 7) + (k2 >> 3), pl.ds((k2 & 7) * 16, 16)] = pv
            return c2
        lax.fori_loop(0, CW // 16, build, 0)

        def scat(j2, c2):
            r = (cb >> 7) + j2
            cp1 = pltpu.async_copy(recb.at[pl.ds(j2 * 128, 128)],
                                   reco.at[posb.at[r]], sem)
            cp2 = pltpu.async_copy(srcb.at[pl.ds(cb + j2 * 128, 128)],
                                   srcso.at[posb.at[r]], sem)
            cp1.wait()
            cp2.wait()
            return c2
        lax.fori_loop(0, CW // 128, scat, 0)
        return c
    lax.fori_loop(0, EPW // CW, chunk, 0)


def _sc_2(stab, srcg, dstg, baset):
    return pl.kernel(
        _sc2_body,
        out_type=[
            jax.ShapeDtypeStruct((EPP, 16), jnp.int32),
            jax.ShapeDtypeStruct((EPP,), jnp.int32),
        ],
        mesh=_mesh(),
        compiler_params=_SC_PARAMS,
        scratch_types=[
            pltpu.VMEM((EPW,), jnp.int32),
            pltpu.VMEM((EPW,), jnp.int32),
            pltpu.VMEM((EPW,), jnp.int32),
            pltpu.VMEM((EPW,), jnp.int32),
            pltpu.VMEM((CW, 16), jnp.float32),
            pltpu.VMEM((CW, 16), jnp.float32),
            pltpu.VMEM((CW, 16), jnp.int32),
            pltpu.VMEM((EPW // 128, 128), jnp.int32),
            pltpu.VMEM((NBP,), jnp.int32),
            pltpu.SMEM((NBP,), jnp.int32),
            pltpu.SMEM((NBP,), jnp.int32),
            pltpu.SemaphoreType.DMA,
        ],
    )(stab, srcg, dstg, baset)


def _sc3_body(heads, xstk, recof, srcso, starts8, zo, deno,
              acc, dacc, xchunk, recc, sidx, startsv, sms, smt, sem):
    wid = _wid()
    iota = _i16(16)
    sh2b = (iota + 2) & 15
    nmax = (2 * N if heads == 8 else N) - 1
    pltpu.sync_copy(starts8.at[0], startsv)

    def tosm(k, c):
        v = startsv[pl.ds(k * 16, 16)]
        for j in range(16):
            sms[k * 16 + j] = v[j]
        return c
    lax.fori_loop(0, NBP // 16, tosm, 0)
    pltpu.sync_copy(starts8.at[1], startsv)

    def tosm2(k, c):
        v = startsv[pl.ds(k * 16, 16)]
        for j in range(16):
            smt[k * 16 + j] = v[j]
        return c
    lax.fori_loop(0, NBP // 16, tosm2, 0)

    ntask = 2 * NB if heads == 8 else NB
    npt = (ntask + NW - 1) // NW

    def task_loop(ti, c):
        task = ti * NW + wid

        @pl.when(task < ntask)
        def _():
            half = jnp.where(task >= NB, 1, 0)
            b = task - half * NB
            sb = pl.multiple_of(sms[b], 8)
            cnt_e = smt[b]

            def za(i, c2):
                for v in range(16):
                    acc[i, pl.ds(v * 16, 16)] = jnp.zeros((16,), jnp.float32)
                return c2
            lax.fori_loop(0, heads * BK, za, 0)

            def zd(i, c2):
                dacc[i, :] = jnp.zeros((16,), jnp.float32)
                return c2
            lax.fori_loop(0, BK, zd, 0)

            nch = (cnt_e + CK3 - 1) >> 7

            def chunk_loop(cc, c2):
                off = pl.multiple_of(sb + cc * CK3, 8)
                pltpu.sync_copy(recof.at[pl.ds(pl.multiple_of(off * 16, 8),
                                               CK3 * 16)], recc)
                pltpu.sync_copy(srcso.at[pl.ds(off, CK3)], sidx)

                def adj(v, c3):
                    sv = sidx[pl.ds(v * 16, 16)] + half * N
                    sidx[pl.ds(v * 16, 16)] = jnp.clip(sv, 0, nmax)
                    return c3
                lax.fori_loop(0, CK3 // 16, adj, 0)
                pltpu.async_copy(xstk.at[sidx], xchunk, sem).wait()
                ecnt = jnp.minimum(CK3, cnt_e - cc * CK3)

                def edge(e, c3):
                    recrow = recc[pl.ds(pl.multiple_of(e * 16, 8), 16)]
                    rowf = lax.bitcast_convert_type(recrow, jnp.float32)
                    dmod = recrow[1] - b * BK

                    @pl.when(half == 0)
                    def _():
                        wv = _perm(rowf, sh2b)
                        wm = jnp.where(iota < heads, wv,
                                       jnp.zeros((16,), jnp.float32))
                        plsc.addupdate(dacc.at[dmod], wm)

                    for hh in range(heads):
                        wsp = _splat(rowf, 2 + hh)
                        arow = hh * BK + dmod
                        for v in range(16):
                            xv = xchunk[e, pl.ds(v * 16, 16)]
                            plsc.addupdate(
                                acc.at[arow, pl.ds(v * 16, 16)], xv * wsp)
                    return c3
                lax.fori_loop(0, ecnt, edge, 0)
                return c2
            lax.fori_loop(0, nch, chunk_loop, 0)

            for hh in range(heads):
                pltpu.sync_copy(
                    acc.at[pl.ds(hh * BK, BK)],
                    zo.at[half * heads + hh, pl.ds(b * BK, BK)])

            @pl.when(half == 0)
            def _():
                pltpu.sync_copy(dacc, deno.at[b])
        return c
    lax.fori_loop(0, npt, task_loop, 0)


def _sc_3(heads, xstk, recof, srcso, starts8):
    body = functools.partial(_sc3_body, heads)
    nz = 2 * heads if heads == 8 else heads
    return pl.kernel(
        body,
        out_type=[
            jax.ShapeDtypeStruct((nz, N, 256), jnp.float32),
            jax.ShapeDtypeStruct((NB, BK, 16), jnp.float32),
        ],
        mesh=_mesh(),
        compiler_params=_SC_PARAMS,
        scratch_types=[
            pltpu.VMEM((heads * BK, 256), jnp.float32),
            pltpu.VMEM((BK, 16), jnp.float32),
            pltpu.VMEM((CK3, 256), jnp.float32),
            pltpu.VMEM((CK3 * 16,), jnp.int32),
            pltpu.VMEM((CK3,), jnp.int32),
            pltpu.VMEM((NBP,), jnp.int32),
            pltpu.SMEM((NBP,), jnp.int32),
            pltpu.SMEM((NBP,), jnp.int32),
            pltpu.SemaphoreType.DMA,
        ],
    )(xstk, recof, srcso, starts8)


def _tca_body(w1r_ref, atts_ref, attd_ref, x_ref, s_ref, wself_ref):
    wa_s = jnp.sum(w1r_ref[...] * atts_ref[...][None], axis=2)
    wa_d = jnp.sum(w1r_ref[...] * attd_ref[...][None], axis=2)
    wa = jnp.concatenate([wa_s, wa_d], axis=1)
    s = x_ref[...] @ wa
    s_ref[...] = s
    wself_ref[...] = jnp.exp(_lrelu(s[:, :H1] + s[:, H1:]))


def _tc_a(x, W1r, atts, attd):
    blk = 1000
    return pl.pallas_call(
        _tca_body,
        grid=(N // blk,),
        in_specs=[
            pl.BlockSpec((D_IN, H1, HID), lambda i: (0, 0, 0)),
            pl.BlockSpec((H1, HID), lambda i: (0, 0)),
            pl.BlockSpec((H1, HID), lambda i: (0, 0)),
            pl.BlockSpec((blk, D_IN), lambda i: (i, 0)),
        ],
        out_specs=[
            pl.BlockSpec((blk, 2 * H1), lambda i: (i, 0)),
            pl.BlockSpec((blk, H1), lambda i: (i, 0)),
        ],
        out_shape=[
            jax.ShapeDtypeStruct((N, 2 * H1), jnp.float32),
            jax.ShapeDtypeStruct((N, H1), jnp.float32),
        ],
    )(W1r, atts, attd, x)


def _tcb_body(z_ref, xp_ref, wself_ref, den_ref, w1p_ref, b1_ref, out_ref):
    ws = wself_ref[0]
    zb = jnp.concatenate([z_ref[0, 0], z_ref[1, 0]], axis=1)
    zb = zb + ws * xp_ref[...]
    o = jnp.dot(zb, w1p_ref[0], preferred_element_type=jnp.float32)
    o = o / (den_ref[0] + ws) + b1_ref[0]
    out_ref[0] = jnp.where(o > 0, o, jnp.exp(jnp.minimum(o, 0.0)) - 1.0)


def _tc_b(z, xp, wselfT3, denT3, W1p, b1r3):
    blk = 1000
    return pl.pallas_call(
        _tcb_body,
        grid=(H1, N // blk),
        in_specs=[
            pl.BlockSpec((2, 1, blk, CH), lambda h, i: (0, h, i, 0)),
            pl.BlockSpec((blk, 2 * CH), lambda h, i: (i, 0)),
            pl.BlockSpec((1, blk, 1), lambda h, i: (h, i, 0)),
            pl.BlockSpec((1, blk, 1), lambda h, i: (h, i, 0)),
            pl.BlockSpec((1, 2 * CH, HID), lambda h, i: (h, 0, 0)),
            pl.BlockSpec((1, 1, HID), lambda h, i: (h, 0, 0)),
        ],
        out_specs=pl.BlockSpec((1, blk, HID), lambda h, i: (h, i, 0)),
        out_shape=jax.ShapeDtypeStruct((H1, N, HID), jnp.float32),
    )(z, xp, wselfT3, denT3, W1p, b1r3)


def _tcc_body(x2_ref, w2_ref, att2_ref, h2_ref, s2_ref):
    k = pl.program_id(1)

    @pl.when(k == 0)
    def _():
        h2_ref[...] = jnp.zeros_like(h2_ref)

    h2_ref[...] += jnp.dot(x2_ref[0], w2_ref[0],
                           preferred_element_type=jnp.float32)

    @pl.when(k == pl.num_programs(1) - 1)
    def _():
        s2_ref[...] = jnp.dot(h2_ref[...], att2_ref[...],
                              preferred_element_type=jnp.float32)


def _tc_c(x2, W2p, att2p):
    blk = 1000
    return pl.pallas_call(
        _tcc_body,
        grid=(N // blk, H1),
        in_specs=[
            pl.BlockSpec((1, blk, HID), lambda i, k: (k, i, 0)),
            pl.BlockSpec((1, HID, CH), lambda i, k: (k, 0, 0)),
            pl.BlockSpec((CH, 2), lambda i, k: (0, 0)),
        ],
        out_specs=[
            pl.BlockSpec((blk, CH), lambda i, k: (i, 0)),
            pl.BlockSpec((blk, 2), lambda i, k: (i, 0)),
        ],
        out_shape=[
            jax.ShapeDtypeStruct((N, CH), jnp.float32),
            jax.ShapeDtypeStruct((N, 2), jnp.float32),
        ],
    )(x2, W2p, att2p)


def _tcd_body(z2_ref, h2_ref, s2_ref, den2_ref, b2_ref, out_ref):
    ws = jnp.exp(_lrelu(s2_ref[:, :1] + s2_ref[:, 1:2]))
    full = (z2_ref[...] + ws * h2_ref[...]) / (den2_ref[...] + ws)
    out_ref[...] = full[:, :OUT] + b2_ref[...]


def _tc_d(z2, h2p, s2, den2, b2):
    blk = 1000
    return pl.pallas_call(
        _tcd_body,
        grid=(N // blk,),
        in_specs=[
            pl.BlockSpec((blk, CH), lambda i: (i, 0)),
            pl.BlockSpec((blk, CH), lambda i: (i, 0)),
            pl.BlockSpec((blk, 2), lambda i: (i, 0)),
            pl.BlockSpec((blk, 1), lambda i: (i, 0)),
            pl.BlockSpec((1, OUT), lambda i: (0, 0)),
        ],
        out_specs=pl.BlockSpec((blk, OUT), lambda i: (i, 0)),
        out_shape=jax.ShapeDtypeStruct((N, OUT), jnp.float32),
    )(z2, h2p, s2, den2, b2)


@jax.jit
def kernel(input_data, edge_index, W1, att_src1, att_dst1, b1,
           W2, att_src2, att_dst2, b2):
    x = input_data

    srcg = jnp.pad(edge_index[0].reshape(NW, E // NW),
                   ((0, 0), (0, EPW - E // NW))).reshape(EP)
    dstg = jnp.pad(edge_index[1].reshape(NW, E // NW),
                   ((0, 0), (0, EPW - E // NW)),
                   constant_values=N).reshape(EP)
    W1r = W1.reshape(D_IN, H1, HID)
    xlo = jnp.pad(x[:, :250], ((0, 0), (0, CH - 250)))
    xhi = jnp.pad(x[:, 250:], ((0, 0), (0, CH - 250)))
    xp = jnp.concatenate([xlo, xhi], axis=1)
    xstk = jnp.concatenate([xlo, xhi], axis=0)
    W1p = jnp.concatenate([
        jnp.pad(W1r[:250], ((0, CH - 250), (0, 0), (0, 0))),
        jnp.pad(W1r[250:], ((0, CH - 250), (0, 0), (0, 0))),
    ], axis=0)
    W1p = jnp.transpose(W1p, (1, 0, 2))
    b1r3 = b1.reshape(H1, 1, HID)
    W2p = jnp.pad(W2, ((0, 0), (0, CH - OUT))).reshape(H1, HID, CH)
    att2p = jnp.pad(
        jnp.stack([att_src2[0, 0], att_dst2[0, 0]], axis=1),
        ((0, CH - OUT), (0, 0)))

    cnt = _sc_h(dstg)
    baset, starts8 = _tc_p(cnt)

    s, wself = _tc_a(x, W1r, att_src1[0], att_dst1[0])
    rec1, srcs1 = _sc_2(s, srcg, dstg, baset)
    z1, den1 = _sc_3(8, xstk, rec1.reshape(EPP * 16), srcs1, starts8)
    z1 = z1.reshape(2, H1, N, CH)
    denom = den1.reshape(N, 16)[:, :H1]
    x2 = _tc_b(z1, xp, wself.T[:, :, None], denom.T[:, :, None],
               W1p, b1r3)

    h2p, s2 = _tc_c(x2, W2p, att2p)
    zcol = jnp.zeros((N, 7), jnp.float32)
    s2pad = jnp.concatenate([s2[:, :1], zcol, s2[:, 1:2], zcol], axis=1)
    rec2, srcs2 = _sc_2(s2pad, srcg, dstg, baset)
    z2, den2 = _sc_3(1, h2p, rec2.reshape(EPP * 16), srcs2, starts8)
    den2v = den2.reshape(N, 16)[:, :1]
    return _tc_d(z2[0], h2p, s2, den2v, b2[None, :])
